# Initial kernel scaffold; baseline (speedup 1.0000x reference)
import jax
import jax.numpy as jnp
from jax.experimental import pallas as pl


def kernel(index, value, field, emb, field_emb, W1, b1, W2, b2):
    raise NotImplementedError("write your pallas kernel here")


# speedup vs baseline: 3.9169x; 3.9169x over previous
import jax
import jax.numpy as jnp
from jax import lax
from jax.experimental import pallas as pl
from jax.experimental.pallas import tpu as pltpu
from jax.experimental.pallas import tpu_sc as plsc

B, L = 4096, 200
HID = 128
NFIELDS = 26
MLP_DIM = 512
LPAD = 208
CHUNK = 104
VPAD = 32


def _sc_pool_kernel(index_hbm, value_hbm, field_hbm, emb_hbm,
                    xe_hbm,
                    idx_v, val_v, fld_v, rows_v, acc_v,
                    sem_in, sem_g, sem_out):
    nc = 2
    wid = lax.axis_index("s") * nc + lax.axis_index("c")
    n_per_w = B // 32
    base = wid * n_per_w

    zeros16i = jnp.zeros((16,), jnp.int32)
    zeros16f = jnp.zeros((16,), jnp.float32)
    idx_v[pl.ds(192, 16)] = zeros16i
    val_v[pl.ds(192, 16)] = zeros16f
    fld_v[pl.ds(192, 16)] = zeros16i

    def one_example(b_local, carry):
        b = base + b_local
        cp_i = pltpu.make_async_copy(
            index_hbm.at[pl.ds(b * L, L)], idx_v.at[pl.ds(0, L)], sem_in)
        cp_v = pltpu.make_async_copy(
            value_hbm.at[pl.ds(b * L, L)], val_v.at[pl.ds(0, L)], sem_in)
        cp_f = pltpu.make_async_copy(
            field_hbm.at[pl.ds(b * L, L)], fld_v.at[pl.ds(0, L)], sem_in)
        cp_i.start(); cp_v.start(); cp_f.start()
        cp_i.wait(); cp_v.wait(); cp_f.wait()

        g0 = pltpu.make_async_copy(
            emb_hbm.at[idx_v.at[pl.ds(0, CHUNK)]], rows_v.at[pl.ds(0, CHUNK)], sem_g)
        g1 = pltpu.make_async_copy(
            emb_hbm.at[idx_v.at[pl.ds(CHUNK, CHUNK)]], rows_v.at[pl.ds(CHUNK, CHUNK)], sem_g)
        g0.start(); g1.start()

        def zero_blk(i, c):
            acc_v[pl.ds(16 * i, 16)] = zeros16f
            return c
        lax.fori_loop(0, NFIELDS * HID // 16, zero_blk, 0, unroll=8)

        g0.wait(); g1.wait()

        def body(t, c):
            fvec = fld_v[pl.ds(16 * t, 16)]
            vvec = val_v[pl.ds(16 * t, 16)]
            for k in range(16):
                f = fvec[k]
                v = vvec[k]
                row = 16 * t + k
                fbase = pl.multiple_of(f * HID, 16)
                for j in range(HID // 16):
                    plsc.addupdate(acc_v.at[pl.ds(fbase + 16 * j, 16)],
                                   v * rows_v[row, pl.ds(16 * j, 16)])
            return c
        lax.fori_loop(0, LPAD // 16, body, 0)

        o0 = pltpu.make_async_copy(
            acc_v, xe_hbm.at[pl.ds(b * NFIELDS * HID, NFIELDS * HID)], sem_out)
        o0.start()
        o0.wait()
        return carry

    lax.fori_loop(0, n_per_w, one_example, 0)


def _sc_pool(index, value, field, emb):
    mesh = plsc.VectorSubcoreMesh(core_axis_name="c", subcore_axis_name="s")
    kern = pl.kernel(
        _sc_pool_kernel,
        mesh=mesh,
        out_type=jax.ShapeDtypeStruct((B * NFIELDS * HID,), jnp.float32),
        scratch_types=[
            pltpu.VMEM((LPAD,), jnp.int32),
            pltpu.VMEM((LPAD,), jnp.float32),
            pltpu.VMEM((LPAD,), jnp.int32),
            pltpu.VMEM((LPAD, HID), jnp.float32),
            pltpu.VMEM((NFIELDS * HID,), jnp.float32),
            pltpu.SemaphoreType.DMA,
            pltpu.SemaphoreType.DMA,
            pltpu.SemaphoreType.DMA,
        ],
    )
    return kern(index.reshape(-1), value.reshape(-1), field.reshape(-1), emb)


def _m_kernel(fe_ref, w1f_ref, m_ref):
    for s in range(NFIELDS):
        m_ref[s, :] = jnp.dot(fe_ref[s, :].reshape(1, HID), w1f_ref[s],
                              preferred_element_type=jnp.float32)[0]


def _mlp_kernel(xe_ref, val_ref, fld_ref, w1e_ref, m_ref, b1_ref, w2_ref,
                b2_ref, out_ref):
    h = jnp.dot(xe_ref[...], w1e_ref[...], preferred_element_type=jnp.float32)
    cols = []
    for s in range(VPAD):
        masked = jnp.where(fld_ref[...] == s, val_ref[...], 0.0)
        cols.append(jnp.sum(masked, axis=1, keepdims=True))
    vs = jnp.concatenate(cols, axis=1)
    h = h + jnp.dot(vs, m_ref[...], preferred_element_type=jnp.float32)
    h = jax.nn.relu(h + b1_ref[...])
    y = jnp.dot(h, w2_ref[...], preferred_element_type=jnp.float32) + b2_ref[0, 0]
    out_ref[...] = y


def kernel(index, value, field, emb, field_emb, W1, b1, W2, b2):
    index = index.astype(jnp.int32)
    field = field.astype(jnp.int32)

    xe = _sc_pool(index, value, field, emb)

    w1r = W1.reshape(NFIELDS, 2, HID, MLP_DIM)
    w1e = w1r[:, 0].reshape(NFIELDS * HID, MLP_DIM)
    w1f = w1r[:, 1]
    fe = field_emb[:NFIELDS]

    m = pl.pallas_call(
        _m_kernel,
        out_shape=jax.ShapeDtypeStruct((NFIELDS, MLP_DIM), jnp.float32),
    )(fe, w1f)
    m_pad = jnp.pad(m, ((0, VPAD - NFIELDS), (0, 0)))

    bt = 256
    grid = (B // bt,)
    out2d = pl.pallas_call(
        _mlp_kernel,
        grid=grid,
        in_specs=[
            pl.BlockSpec((bt, NFIELDS * HID), lambda i: (i, 0)),
            pl.BlockSpec((bt, L), lambda i: (i, 0)),
            pl.BlockSpec((bt, L), lambda i: (i, 0)),
            pl.BlockSpec((NFIELDS * HID, MLP_DIM), lambda i: (0, 0)),
            pl.BlockSpec((VPAD, MLP_DIM), lambda i: (0, 0)),
            pl.BlockSpec((1, MLP_DIM), lambda i: (0, 0)),
            pl.BlockSpec((MLP_DIM, 1), lambda i: (0, 0)),
            pl.BlockSpec((1, 1), lambda i: (0, 0)),
        ],
        out_specs=pl.BlockSpec((bt, 1), lambda i: (i, 0)),
        out_shape=jax.ShapeDtypeStruct((B, 1), jnp.float32),
    )(xe.reshape(B, NFIELDS * HID), value, field, w1e, m_pad,
      b1.reshape(1, MLP_DIM), W2, b2.reshape(1, 1))

    return out2d[:, 0]


# speedup vs baseline: 5.5527x; 1.4176x over previous
import jax
import jax.numpy as jnp
from jax import lax
from jax.experimental import pallas as pl
from jax.experimental.pallas import tpu as pltpu
from jax.experimental.pallas import tpu_sc as plsc

B, L = 4096, 200
HID = 128
NFIELDS = 26
MLP_DIM = 512
VPAD = 32
NW = 32
EPC = 64
GC0, GC1 = 104, 96
ACC = NFIELDS * HID


def _sc_pool_kernel(index_hbm, value_hbm, field_hbm, emb_hbm,
                    xe_hbm,
                    idx_v, val_v, fld_v, rows_a, rows_b, acc_a, acc_b,
                    sem_in, sem_ga, sem_gb, sem_oa, sem_ob):
    nc = 2
    wid = lax.axis_index("s") * nc + lax.axis_index("c")
    n_per_w = B // NW
    base = wid * n_per_w

    zeros16f = jnp.zeros((16,), jnp.float32)
    iota16 = lax.iota(jnp.int32, 16)

    def g_start(e, rows_ref, sem):
        off = e * L
        pltpu.make_async_copy(
            emb_hbm.at[idx_v.at[pl.ds(off, GC0)]],
            rows_ref.at[pl.ds(0, GC0)], sem).start()
        pltpu.make_async_copy(
            emb_hbm.at[idx_v.at[pl.ds(off + GC0, GC1)]],
            rows_ref.at[pl.ds(GC0, GC1)], sem).start()

    def g_wait(e, rows_ref, sem):
        off = e * L
        pltpu.make_async_copy(
            emb_hbm.at[idx_v.at[pl.ds(off, GC0)]],
            rows_ref.at[pl.ds(0, GC0)], sem).wait()
        pltpu.make_async_copy(
            emb_hbm.at[idx_v.at[pl.ds(off + GC0, GC1)]],
            rows_ref.at[pl.ds(GC0, GC1)], sem).wait()

    def wb_start(b, acc_ref, sem):
        pltpu.make_async_copy(acc_ref, xe_hbm.at[pl.ds(b * ACC, ACC)], sem).start()

    def wb_wait(b, acc_ref, sem):
        pltpu.make_async_copy(acc_ref, xe_hbm.at[pl.ds(b * ACC, ACC)], sem).wait()

    def compute(e, rows_ref, acc_ref):
        def zero_blk(i, c):
            acc_ref[pl.ds(16 * i, 16)] = zeros16f
            return c
        lax.fori_loop(0, ACC // 16, zero_blk, 0, unroll=8)

        off = e * L

        def body(l, c):
            pos = jnp.full((16,), off + l, jnp.int32)
            fs = plsc.load_gather(fld_v, [pos])
            vs = plsc.load_gather(val_v, [pos])
            addr = fs * HID + iota16
            for j in range(HID // 16):
                x = vs * rows_ref[l, pl.ds(16 * j, 16)]
                plsc.addupdate_scatter(acc_ref, [addr + 16 * j], x)
            return c
        lax.fori_loop(0, L, body, 0, unroll=2)

    for c in range(n_per_w // EPC):
        cb = base + c * EPC
        cp_i = pltpu.make_async_copy(
            index_hbm.at[pl.ds(cb * L, EPC * L)], idx_v, sem_in)
        cp_v = pltpu.make_async_copy(
            value_hbm.at[pl.ds(cb * L, EPC * L)], val_v, sem_in)
        cp_f = pltpu.make_async_copy(
            field_hbm.at[pl.ds(cb * L, EPC * L)], fld_v, sem_in)
        cp_i.start(); cp_v.start(); cp_f.start()
        cp_i.wait(); cp_v.wait(); cp_f.wait()

        g_start(0, rows_a, sem_ga)

        def pair(ep, carry):
            e0 = 2 * ep
            e1 = 2 * ep + 1
            g_start(e1, rows_b, sem_gb)

            @pl.when(ep > 0)
            def _():
                wb_wait(cb + e0 - 2, acc_a, sem_oa)
            g_wait(e0, rows_a, sem_ga)
            compute(e0, rows_a, acc_a)
            wb_start(cb + e0, acc_a, sem_oa)

            @pl.when(ep < EPC // 2 - 1)
            def _():
                g_start(e1 + 1, rows_a, sem_ga)

            @pl.when(ep > 0)
            def _():
                wb_wait(cb + e1 - 2, acc_b, sem_ob)
            g_wait(e1, rows_b, sem_gb)
            compute(e1, rows_b, acc_b)
            wb_start(cb + e1, acc_b, sem_ob)
            return carry

        lax.fori_loop(0, EPC // 2, pair, 0)
        wb_wait(cb + EPC - 2, acc_a, sem_oa)
        wb_wait(cb + EPC - 1, acc_b, sem_ob)


def _sc_pool(index, value, field, emb):
    mesh = plsc.VectorSubcoreMesh(core_axis_name="c", subcore_axis_name="s")
    kern = pl.kernel(
        _sc_pool_kernel,
        mesh=mesh,
        compiler_params=pltpu.CompilerParams(needs_layout_passes=False),
        out_type=jax.ShapeDtypeStruct((B * ACC,), jnp.float32),
        scratch_types=[
            pltpu.VMEM((EPC * L,), jnp.int32),
            pltpu.VMEM((EPC * L,), jnp.float32),
            pltpu.VMEM((EPC * L,), jnp.int32),
            pltpu.VMEM((L, HID), jnp.float32),
            pltpu.VMEM((L, HID), jnp.float32),
            pltpu.VMEM((ACC,), jnp.float32),
            pltpu.VMEM((ACC,), jnp.float32),
            pltpu.SemaphoreType.DMA,
            pltpu.SemaphoreType.DMA,
            pltpu.SemaphoreType.DMA,
            pltpu.SemaphoreType.DMA,
            pltpu.SemaphoreType.DMA,
        ],
    )
    return kern(index.reshape(-1), value.reshape(-1), field.reshape(-1), emb)


def _m_kernel(fe_ref, w1f_ref, m_ref):
    for s in range(NFIELDS):
        m_ref[s, :] = jnp.dot(fe_ref[s, :].reshape(1, HID), w1f_ref[s],
                              preferred_element_type=jnp.float32)[0]


def _mlp_kernel(xe_ref, val_ref, fld_ref, w1e_ref, m_ref, b1_ref, w2_ref,
                b2_ref, out_ref):
    h = jnp.dot(xe_ref[...], w1e_ref[...], preferred_element_type=jnp.float32)
    cols = []
    for s in range(VPAD):
        masked = jnp.where(fld_ref[...] == s, val_ref[...], 0.0)
        cols.append(jnp.sum(masked, axis=1, keepdims=True))
    vs = jnp.concatenate(cols, axis=1)
    h = h + jnp.dot(vs, m_ref[...], preferred_element_type=jnp.float32)
    h = jax.nn.relu(h + b1_ref[...])
    y = jnp.dot(h, w2_ref[...], preferred_element_type=jnp.float32) + b2_ref[0, 0]
    out_ref[...] = y


def kernel(index, value, field, emb, field_emb, W1, b1, W2, b2):
    index = index.astype(jnp.int32)
    field = field.astype(jnp.int32)

    xe = _sc_pool(index, value, field, emb)

    w1r = W1.reshape(NFIELDS, 2, HID, MLP_DIM)
    w1e = w1r[:, 0].reshape(NFIELDS * HID, MLP_DIM)
    w1f = w1r[:, 1]
    fe = field_emb[:NFIELDS]

    m = pl.pallas_call(
        _m_kernel,
        out_shape=jax.ShapeDtypeStruct((NFIELDS, MLP_DIM), jnp.float32),
    )(fe, w1f)
    m_pad = jnp.pad(m, ((0, VPAD - NFIELDS), (0, 0)))

    bt = 256
    grid = (B // bt,)
    out2d = pl.pallas_call(
        _mlp_kernel,
        grid=grid,
        in_specs=[
            pl.BlockSpec((bt, NFIELDS * HID), lambda i: (i, 0)),
            pl.BlockSpec((bt, L), lambda i: (i, 0)),
            pl.BlockSpec((bt, L), lambda i: (i, 0)),
            pl.BlockSpec((NFIELDS * HID, MLP_DIM), lambda i: (0, 0)),
            pl.BlockSpec((VPAD, MLP_DIM), lambda i: (0, 0)),
            pl.BlockSpec((1, MLP_DIM), lambda i: (0, 0)),
            pl.BlockSpec((MLP_DIM, 1), lambda i: (0, 0)),
            pl.BlockSpec((1, 1), lambda i: (0, 0)),
        ],
        out_specs=pl.BlockSpec((bt, 1), lambda i: (i, 0)),
        out_shape=jax.ShapeDtypeStruct((B, 1), jnp.float32),
    )(xe.reshape(B, NFIELDS * HID), value, field, w1e, m_pad,
      b1.reshape(1, MLP_DIM), W2, b2.reshape(1, 1))

    return out2d[:, 0]


# speedup vs baseline: 14.7616x; 2.6585x over previous
import jax
import jax.numpy as jnp
from jax import lax
from jax.experimental import pallas as pl
from jax.experimental.pallas import tpu as pltpu
from jax.experimental.pallas import tpu_sc as plsc

B, L = 4096, 200
HID = 128
NFIELDS = 26
MLP_DIM = 512
VPAD = 32
NW = 32
EPC = 64
GC0, GC1 = 104, 96
ACC = NFIELDS * HID


def _sc_pool_kernel(index_hbm, value_hbm, field_hbm, emb_hbm,
                    xe_hbm,
                    idx_v, val_v, fld_v, rows_a, rows_b, acc_a, acc_b,
                    sem_in, sem_ga, sem_gb, sem_oa, sem_ob):
    nc = 2
    wid = lax.axis_index("s") * nc + lax.axis_index("c")
    n_per_w = B // NW
    base = wid * n_per_w

    zeros16f = jnp.zeros((16,), jnp.float32)
    iota16 = lax.iota(jnp.int32, 16)

    def g_start(e, rows_ref, sem):
        off = e * L
        pltpu.make_async_copy(
            emb_hbm.at[idx_v.at[pl.ds(off, GC0)]],
            rows_ref.at[pl.ds(0, GC0)], sem).start()
        pltpu.make_async_copy(
            emb_hbm.at[idx_v.at[pl.ds(off + GC0, GC1)]],
            rows_ref.at[pl.ds(GC0, GC1)], sem).start()

    def g_wait(e, rows_ref, sem):
        off = e * L
        pltpu.make_async_copy(
            emb_hbm.at[idx_v.at[pl.ds(off, GC0)]],
            rows_ref.at[pl.ds(0, GC0)], sem).wait()
        pltpu.make_async_copy(
            emb_hbm.at[idx_v.at[pl.ds(off + GC0, GC1)]],
            rows_ref.at[pl.ds(GC0, GC1)], sem).wait()

    def wb_start(b, acc_ref, sem):
        pltpu.make_async_copy(acc_ref, xe_hbm.at[pl.ds(b * ACC, ACC)], sem).start()

    def wb_wait(b, acc_ref, sem):
        pltpu.make_async_copy(acc_ref, xe_hbm.at[pl.ds(b * ACC, ACC)], sem).wait()

    def compute(e, rows_ref, acc_ref):
        @plsc.parallel_loop(0, ACC // 16, unroll=8)
        def _(i):
            acc_ref[pl.ds(16 * i, 16)] = zeros16f

        off = e * L

        @plsc.parallel_loop(0, L, unroll=4)
        def _(l):
            pos = jnp.full((16,), off + l, jnp.int32)
            fs = plsc.load_gather(fld_v, [pos])
            vs = plsc.load_gather(val_v, [pos])
            addr = fs * HID + iota16
            for j in range(HID // 16):
                x = vs * rows_ref[l, pl.ds(16 * j, 16)]
                plsc.addupdate_scatter(acc_ref, [addr + 16 * j], x)

    for c in range(n_per_w // EPC):
        cb = base + c * EPC
        cp_i = pltpu.make_async_copy(
            index_hbm.at[pl.ds(cb * L, EPC * L)], idx_v, sem_in)
        cp_v = pltpu.make_async_copy(
            value_hbm.at[pl.ds(cb * L, EPC * L)], val_v, sem_in)
        cp_f = pltpu.make_async_copy(
            field_hbm.at[pl.ds(cb * L, EPC * L)], fld_v, sem_in)
        cp_i.start(); cp_v.start(); cp_f.start()
        cp_i.wait(); cp_v.wait(); cp_f.wait()

        g_start(0, rows_a, sem_ga)

        def pair(ep, carry):
            e0 = 2 * ep
            e1 = 2 * ep + 1
            g_start(e1, rows_b, sem_gb)

            @pl.when(ep > 0)
            def _():
                wb_wait(cb + e0 - 2, acc_a, sem_oa)
            g_wait(e0, rows_a, sem_ga)
            compute(e0, rows_a, acc_a)
            wb_start(cb + e0, acc_a, sem_oa)

            @pl.when(ep < EPC // 2 - 1)
            def _():
                g_start(e1 + 1, rows_a, sem_ga)

            @pl.when(ep > 0)
            def _():
                wb_wait(cb + e1 - 2, acc_b, sem_ob)
            g_wait(e1, rows_b, sem_gb)
            compute(e1, rows_b, acc_b)
            wb_start(cb + e1, acc_b, sem_ob)
            return carry

        lax.fori_loop(0, EPC // 2, pair, 0)
        wb_wait(cb + EPC - 2, acc_a, sem_oa)
        wb_wait(cb + EPC - 1, acc_b, sem_ob)


def _sc_pool(index, value, field, emb):
    mesh = plsc.VectorSubcoreMesh(core_axis_name="c", subcore_axis_name="s")
    kern = pl.kernel(
        _sc_pool_kernel,
        mesh=mesh,
        compiler_params=pltpu.CompilerParams(needs_layout_passes=False),
        out_type=jax.ShapeDtypeStruct((B * ACC,), jnp.float32),
        scratch_types=[
            pltpu.VMEM((EPC * L,), jnp.int32),
            pltpu.VMEM((EPC * L,), jnp.float32),
            pltpu.VMEM((EPC * L,), jnp.int32),
            pltpu.VMEM((L, HID), jnp.float32),
            pltpu.VMEM((L, HID), jnp.float32),
            pltpu.VMEM((ACC,), jnp.float32),
            pltpu.VMEM((ACC,), jnp.float32),
            pltpu.SemaphoreType.DMA,
            pltpu.SemaphoreType.DMA,
            pltpu.SemaphoreType.DMA,
            pltpu.SemaphoreType.DMA,
            pltpu.SemaphoreType.DMA,
        ],
    )
    return kern(index.reshape(-1), value.reshape(-1), field.reshape(-1), emb)


def _m_kernel(fe_ref, w1f_ref, m_ref):
    for s in range(NFIELDS):
        m_ref[s, :] = jnp.dot(fe_ref[s, :].reshape(1, HID), w1f_ref[s],
                              preferred_element_type=jnp.float32)[0]


def _mlp_kernel(xe_ref, val_ref, fld_ref, w1e_ref, m_ref, b1_ref, w2_ref,
                b2_ref, out_ref):
    h = jnp.dot(xe_ref[...], w1e_ref[...], preferred_element_type=jnp.float32)
    cols = []
    for s in range(VPAD):
        masked = jnp.where(fld_ref[...] == s, val_ref[...], 0.0)
        cols.append(jnp.sum(masked, axis=1, keepdims=True))
    vs = jnp.concatenate(cols, axis=1)
    h = h + jnp.dot(vs, m_ref[...], preferred_element_type=jnp.float32)
    h = jax.nn.relu(h + b1_ref[...])
    y = jnp.dot(h, w2_ref[...], preferred_element_type=jnp.float32) + b2_ref[0, 0]
    out_ref[...] = y


def kernel(index, value, field, emb, field_emb, W1, b1, W2, b2):
    index = index.astype(jnp.int32)
    field = field.astype(jnp.int32)

    xe = _sc_pool(index, value, field, emb)

    w1r = W1.reshape(NFIELDS, 2, HID, MLP_DIM)
    w1e = w1r[:, 0].reshape(NFIELDS * HID, MLP_DIM)
    w1f = w1r[:, 1]
    fe = field_emb[:NFIELDS]

    m = pl.pallas_call(
        _m_kernel,
        out_shape=jax.ShapeDtypeStruct((NFIELDS, MLP_DIM), jnp.float32),
    )(fe, w1f)
    m_pad = jnp.pad(m, ((0, VPAD - NFIELDS), (0, 0)))

    bt = 256
    grid = (B // bt,)
    out2d = pl.pallas_call(
        _mlp_kernel,
        grid=grid,
        in_specs=[
            pl.BlockSpec((bt, NFIELDS * HID), lambda i: (i, 0)),
            pl.BlockSpec((bt, L), lambda i: (i, 0)),
            pl.BlockSpec((bt, L), lambda i: (i, 0)),
            pl.BlockSpec((NFIELDS * HID, MLP_DIM), lambda i: (0, 0)),
            pl.BlockSpec((VPAD, MLP_DIM), lambda i: (0, 0)),
            pl.BlockSpec((1, MLP_DIM), lambda i: (0, 0)),
            pl.BlockSpec((MLP_DIM, 1), lambda i: (0, 0)),
            pl.BlockSpec((1, 1), lambda i: (0, 0)),
        ],
        out_specs=pl.BlockSpec((bt, 1), lambda i: (i, 0)),
        out_shape=jax.ShapeDtypeStruct((B, 1), jnp.float32),
    )(xe.reshape(B, NFIELDS * HID), value, field, w1e, m_pad,
      b1.reshape(1, MLP_DIM), W2, b2.reshape(1, 1))

    return out2d[:, 0]
